# baseline (speedup 1.0000x reference)
import functools

import jax
import jax.numpy as jnp
from jax import lax
from jax.experimental import pallas as pl
from jax.experimental.pallas import tpu as pltpu
from jax.experimental.pallas import tpu_sc as plsc

_L = 768
_D = 256
_K = 8192
_Tb = 256
_Kb = 2048
_NKB = _K // _Kb
_EPS = 1e-8


def _vq_body(tok_ref, w_ref, b_ref, cb_ref, idx_ref, s_ref,
             cbn_ref, sel_ref):
    i = pl.program_id(0)

    @pl.when(i == 0)
    def _():
        cb = cb_ref[...]
        nrm = jnp.sqrt(jnp.sum(cb * cb, axis=1, keepdims=True))
        cbn_ref[...] = cb / (nrm + _EPS)
        col = lax.broadcasted_iota(jnp.int32, (_K, 128), 1)
        row = lax.broadcasted_iota(jnp.int32, (_K, 128), 0).astype(jnp.float32)
        sel_ref[...] = jnp.where(col == 0, row,
                                 jnp.where(col == 1, 1.0, 0.0))

    h = jnp.dot(tok_ref[...], w_ref[...],
                preferred_element_type=jnp.float32) + b_ref[...]
    nrm = jnp.sqrt(jnp.sum(h * h, axis=1, keepdims=True))
    xn = h / (nrm + _EPS)

    sim = lax.dot_general(
        xn, cbn_ref[...],
        (((1,), (1,)), ((), ())), preferred_element_type=jnp.float32)

    m = jnp.max(sim, axis=1, keepdims=True)
    oh = jnp.where(sim >= m, 1.0, 0.0)
    mm = jnp.dot(oh, sel_ref[...], preferred_element_type=jnp.float32)
    cnt = mm[:, 1:2]
    idx_ref[...] = mm[:, 0:1].astype(jnp.int32).reshape(idx_ref.shape)

    @pl.when(jnp.max(cnt) > 1.5)
    def _():
        iota = lax.broadcasted_iota(
            jnp.int32, (_Tb, _K), 1).astype(jnp.float32)
        aml = jnp.min(jnp.where(sim >= m, iota, float(_K)),
                      axis=1, keepdims=True)
        idx_ref[...] = aml.astype(jnp.int32).reshape(idx_ref.shape)

    tot = jnp.sum(m).reshape(1, 1)

    @pl.when(i == 0)
    def _():
        s_ref[...] = tot

    @pl.when(i != 0)
    def _():
        s_ref[...] = s_ref[...] + tot

    @pl.when(i == pl.num_programs(0) - 1)
    def _():
        n_tok = pl.num_programs(0) * _Tb
        scale = 0.25 * 2.0 / (n_tok * _D)
        s_ref[...] = scale * (n_tok - s_ref[...])


def _vq_head(tokens, W, b, cb):
    n = tokens.shape[0]
    grid_i = n // _Tb
    idx3, s = pl.pallas_call(
        _vq_body,
        grid=(grid_i,),
        in_specs=[
            pl.BlockSpec((_Tb, _L), lambda i: (i, 0)),
            pl.BlockSpec((_L, _D), lambda i: (0, 0)),
            pl.BlockSpec((1, _D), lambda i: (0, 0)),
            pl.BlockSpec((_K, _D), lambda i: (0, 0)),
        ],
        out_specs=[
            pl.BlockSpec((1, 1, _Tb), lambda i: (i, 0, 0)),
            pl.BlockSpec((1, 1), lambda i: (0, 0)),
        ],
        out_shape=[
            jax.ShapeDtypeStruct((grid_i, 1, _Tb), jnp.int32),
            jax.ShapeDtypeStruct((1, 1), jnp.float32),
        ],
        scratch_shapes=[
            pltpu.VMEM((_K, _D), jnp.float32),
            pltpu.VMEM((_K, 128), jnp.float32),
        ],
    )(tokens, W, b.reshape(1, _D), cb)
    return idx3.reshape(-1), s[0, 0]


def _sc_gather2(cb_i, cb_d, idx_i, idx_d):
    info = plsc.get_sparse_core_info()
    nw = info.num_cores * info.num_subcores
    n = idx_i.shape[0]
    bpw = n // nw
    mesh = plsc.VectorSubcoreMesh(core_axis_name="c", subcore_axis_name="s")

    def body(cbi_hbm, cbd_hbm, idxi_hbm, idxd_hbm, qi_hbm, qd_hbm,
             iv1, rv1, iv2, rv2, sem):
        wid = lax.axis_index("s") * info.num_cores + lax.axis_index("c")
        base = wid * bpw
        pltpu.sync_copy(idxi_hbm.at[pl.ds(base, bpw)], iv1)
        h1 = pltpu.async_copy(cbi_hbm.at[iv1], rv1, sem)
        pltpu.sync_copy(idxd_hbm.at[pl.ds(base, bpw)], iv2)
        h2 = pltpu.async_copy(cbd_hbm.at[iv2], rv2, sem)
        h1.wait()
        pltpu.sync_copy(rv1, qi_hbm.at[pl.ds(base, bpw)])
        h2.wait()
        pltpu.sync_copy(rv2, qd_hbm.at[pl.ds(base, bpw)])

    return pl.kernel(
        body, mesh=mesh,
        out_type=[jax.ShapeDtypeStruct((n, _D), jnp.float32),
                  jax.ShapeDtypeStruct((n, _D), jnp.float32)],
        scratch_types=[
            pltpu.VMEM((bpw,), jnp.int32),
            pltpu.VMEM((bpw, _D), jnp.float32),
            pltpu.VMEM((bpw,), jnp.int32),
            pltpu.VMEM((bpw, _D), jnp.float32),
            pltpu.SemaphoreType.DMA,
        ],
    )(cb_i, cb_d, idx_i, idx_d)


def _dec_body(qi_ref, qd_ref, wd_ref, bd_ref, out_ref):
    qi = qi_ref[...]
    qi = qi / (jnp.sqrt(jnp.sum(qi * qi, axis=1, keepdims=True)) + _EPS)
    qd = qd_ref[...]
    qd = qd / (jnp.sqrt(jnp.sum(qd * qd, axis=1, keepdims=True)) + _EPS)
    acc = jnp.dot(qi, wd_ref[0:_D, :], preferred_element_type=jnp.float32)
    acc = acc + jnp.dot(qd, wd_ref[_D:2 * _D, :],
                        preferred_element_type=jnp.float32)
    out_ref[...] = acc + bd_ref[...]


def _decoder(q_inv, q_dep, W_dec, b_dec):
    n = q_inv.shape[0]
    blk = 512
    return pl.pallas_call(
        _dec_body,
        grid=(n // blk,),
        in_specs=[
            pl.BlockSpec((blk, _D), lambda i: (i, 0)),
            pl.BlockSpec((blk, _D), lambda i: (i, 0)),
            pl.BlockSpec((2 * _D, _L), lambda i: (0, 0)),
            pl.BlockSpec((1, _L), lambda i: (0, 0)),
        ],
        out_specs=pl.BlockSpec((blk, _L), lambda i: (i, 0)),
        out_shape=jax.ShapeDtypeStruct((n, _L), jnp.float32),
    )(q_inv, q_dep, W_dec, b_dec.reshape(1, _L))


def kernel(h_inv_tokens, h_dep_tokens, W_inv, b_inv, W_dep, b_dep,
           cb_inv, cb_dep, W_dec, b_dec):
    B, T, L = h_inv_tokens.shape
    n = B * T
    ti = h_inv_tokens.reshape(n, L)
    td = h_dep_tokens.reshape(n, L)

    idx_i, loss_i = _vq_head(ti, W_inv, b_inv, cb_inv)
    idx_d, loss_d = _vq_head(td, W_dep, b_dep, cb_dep)
    q_i, q_d = _sc_gather2(cb_inv, cb_dep, idx_i, idx_d)

    z = _decoder(q_i, q_d, W_dec, b_dec).reshape(B, T, L)
    return z, loss_i, loss_d, idx_i.reshape(B, T), idx_d.reshape(B, T)


# speedup vs baseline: 1.7785x; 1.7785x over previous
import functools

import jax
import jax.numpy as jnp
from jax import lax
from jax.experimental import pallas as pl
from jax.experimental.pallas import tpu as pltpu
from jax.experimental.pallas import tpu_sc as plsc

_L = 768
_D = 256
_K = 8192
_Tb = 512
_Kb = 2048
_NKB = _K // _Kb
_EPS = 1e-8


def _vq_body(tok_ref, w_ref, b_ref, cb_ref, idx_ref, s_ref,
             cbn_ref, xn_ref, sim_ref, m_ref):
    i = pl.program_id(0)

    @pl.when(i == 0)
    def _():
        cb = cb_ref[...]
        nrm = jnp.sqrt(jnp.sum(cb * cb, axis=1, keepdims=True))
        cbn_ref[...] = cb / (nrm + _EPS)

    h = jnp.dot(tok_ref[...], w_ref[...],
                preferred_element_type=jnp.float32) + b_ref[...]
    nrm = jnp.sqrt(jnp.sum(h * h, axis=1, keepdims=True))
    xn_ref[...] = h / (nrm + _EPS)

    sim_ref[...] = lax.dot_general(
        xn_ref[...], cbn_ref[...],
        (((1,), (1,)), ((), ())), preferred_element_type=jnp.float32)

    m_ref[...] = jnp.max(sim_ref[...], axis=1, keepdims=True)
    m = m_ref[...]
    aml = jnp.argmax(sim_ref[...], axis=1).astype(jnp.int32)
    idx_ref[...] = aml.reshape(idx_ref.shape)

    tot = jnp.sum(m).reshape(1, 1)

    @pl.when(i == 0)
    def _():
        s_ref[...] = tot

    @pl.when(i != 0)
    def _():
        s_ref[...] = s_ref[...] + tot

    @pl.when(i == pl.num_programs(0) - 1)
    def _():
        n_tok = pl.num_programs(0) * _Tb
        scale = 0.25 * 2.0 / (n_tok * _D)
        s_ref[...] = scale * (n_tok - s_ref[...])


def _vq_head(tokens, W, b, cb):
    n = tokens.shape[0]
    grid_i = n // _Tb
    idx3, s = pl.pallas_call(
        _vq_body,
        grid=(grid_i,),
        in_specs=[
            pl.BlockSpec((_Tb, _L), lambda i: (i, 0)),
            pl.BlockSpec((_L, _D), lambda i: (0, 0)),
            pl.BlockSpec((1, _D), lambda i: (0, 0)),
            pl.BlockSpec((_K, _D), lambda i: (0, 0)),
        ],
        out_specs=[
            pl.BlockSpec((1, 1, _Tb), lambda i: (i, 0, 0)),
            pl.BlockSpec((1, 1), lambda i: (0, 0)),
        ],
        out_shape=[
            jax.ShapeDtypeStruct((grid_i, 1, _Tb), jnp.int32),
            jax.ShapeDtypeStruct((1, 1), jnp.float32),
        ],
        scratch_shapes=[
            pltpu.VMEM((_K, _D), jnp.float32),
            pltpu.VMEM((_Tb, _D), jnp.float32),
            pltpu.VMEM((_Tb, _K), jnp.float32),
            pltpu.VMEM((_Tb, 1), jnp.float32),
        ],
    )(tokens, W, b.reshape(1, _D), cb)
    return idx3.reshape(-1), s[0, 0]


def _sc_gather2(cb_i, cb_d, idx_i, idx_d):
    info = plsc.get_sparse_core_info()
    nw = info.num_cores * info.num_subcores
    n = idx_i.shape[0]
    bpw = n // nw
    mesh = plsc.VectorSubcoreMesh(core_axis_name="c", subcore_axis_name="s")

    def body(cbi_hbm, cbd_hbm, idxi_hbm, idxd_hbm, qi_hbm, qd_hbm,
             iv1, rv1, iv2, rv2, sem):
        wid = lax.axis_index("s") * info.num_cores + lax.axis_index("c")
        base = wid * bpw
        pltpu.sync_copy(idxi_hbm.at[pl.ds(base, bpw)], iv1)
        h1 = pltpu.async_copy(cbi_hbm.at[iv1], rv1, sem)
        pltpu.sync_copy(idxd_hbm.at[pl.ds(base, bpw)], iv2)
        h2 = pltpu.async_copy(cbd_hbm.at[iv2], rv2, sem)
        h1.wait()
        pltpu.sync_copy(rv1, qi_hbm.at[pl.ds(base, bpw)])
        h2.wait()
        pltpu.sync_copy(rv2, qd_hbm.at[pl.ds(base, bpw)])

    return pl.kernel(
        body, mesh=mesh,
        out_type=[jax.ShapeDtypeStruct((n, _D), jnp.float32),
                  jax.ShapeDtypeStruct((n, _D), jnp.float32)],
        scratch_types=[
            pltpu.VMEM((bpw,), jnp.int32),
            pltpu.VMEM((bpw, _D), jnp.float32),
            pltpu.VMEM((bpw,), jnp.int32),
            pltpu.VMEM((bpw, _D), jnp.float32),
            pltpu.SemaphoreType.DMA,
        ],
    )(cb_i, cb_d, idx_i, idx_d)


def _dec_body(qi_ref, qd_ref, wd_ref, bd_ref, out_ref):
    qi = qi_ref[...]
    qi = qi / (jnp.sqrt(jnp.sum(qi * qi, axis=1, keepdims=True)) + _EPS)
    qd = qd_ref[...]
    qd = qd / (jnp.sqrt(jnp.sum(qd * qd, axis=1, keepdims=True)) + _EPS)
    acc = jnp.dot(qi, wd_ref[0:_D, :], preferred_element_type=jnp.float32)
    acc = acc + jnp.dot(qd, wd_ref[_D:2 * _D, :],
                        preferred_element_type=jnp.float32)
    out_ref[...] = acc + bd_ref[...]


def _decoder(q_inv, q_dep, W_dec, b_dec):
    n = q_inv.shape[0]
    blk = 512
    return pl.pallas_call(
        _dec_body,
        grid=(n // blk,),
        in_specs=[
            pl.BlockSpec((blk, _D), lambda i: (i, 0)),
            pl.BlockSpec((blk, _D), lambda i: (i, 0)),
            pl.BlockSpec((2 * _D, _L), lambda i: (0, 0)),
            pl.BlockSpec((1, _L), lambda i: (0, 0)),
        ],
        out_specs=pl.BlockSpec((blk, _L), lambda i: (i, 0)),
        out_shape=jax.ShapeDtypeStruct((n, _L), jnp.float32),
    )(q_inv, q_dep, W_dec, b_dec.reshape(1, _L))


def kernel(h_inv_tokens, h_dep_tokens, W_inv, b_inv, W_dep, b_dep,
           cb_inv, cb_dep, W_dec, b_dec):
    B, T, L = h_inv_tokens.shape
    n = B * T
    ti = h_inv_tokens.reshape(n, L)
    td = h_dep_tokens.reshape(n, L)

    idx_i, loss_i = _vq_head(ti, W_inv, b_inv, cb_inv)
    idx_d, loss_d = _vq_head(td, W_dep, b_dep, cb_dep)
    q_i, q_d = _sc_gather2(cb_inv, cb_dep, idx_i, idx_d)

    z = _decoder(q_i, q_d, W_dec, b_dec).reshape(B, T, L)
    return z, loss_i, loss_d, idx_i.reshape(B, T), idx_d.reshape(B, T)


# speedup vs baseline: 1.9932x; 1.1207x over previous
import functools

import jax
import jax.numpy as jnp
from jax import lax
from jax.experimental import pallas as pl
from jax.experimental.pallas import tpu as pltpu
from jax.experimental.pallas import tpu_sc as plsc

_L = 768
_D = 256
_K = 8192
_Tb = 512
_Kb = 2048
_NKB = _K // _Kb
_EPS = 1e-8


def _vq_body(tok_ref, w_ref, b_ref, cb_ref, idx_ref, xn_out_ref,
             cbn_ref, xn_ref, sim_ref):
    i = pl.program_id(0)

    @pl.when(i == 0)
    def _():
        cb = cb_ref[...]
        nrm = jnp.sqrt(jnp.sum(cb * cb, axis=1, keepdims=True))
        cbn_ref[...] = cb / (nrm + _EPS)

    h = jnp.dot(tok_ref[...], w_ref[...],
                preferred_element_type=jnp.float32) + b_ref[...]
    nrm = jnp.sqrt(jnp.sum(h * h, axis=1, keepdims=True))
    xn_ref[...] = h / (nrm + _EPS)
    xn_out_ref[...] = xn_ref[...]

    sim_ref[...] = lax.dot_general(
        xn_ref[...], cbn_ref[...],
        (((1,), (1,)), ((), ())), preferred_element_type=jnp.float32)

    aml = jnp.argmax(sim_ref[...], axis=1).astype(jnp.int32)
    idx_ref[...] = aml.reshape(idx_ref.shape)


def _vq_head(tokens, W, b, cb):
    n = tokens.shape[0]
    grid_i = n // _Tb
    idx3, xn = pl.pallas_call(
        _vq_body,
        grid=(grid_i,),
        in_specs=[
            pl.BlockSpec((_Tb, _L), lambda i: (i, 0)),
            pl.BlockSpec((_L, _D), lambda i: (0, 0)),
            pl.BlockSpec((1, _D), lambda i: (0, 0)),
            pl.BlockSpec((_K, _D), lambda i: (0, 0)),
        ],
        out_specs=[
            pl.BlockSpec((1, 1, _Tb), lambda i: (i, 0, 0)),
            pl.BlockSpec((_Tb, _D), lambda i: (i, 0)),
        ],
        out_shape=[
            jax.ShapeDtypeStruct((grid_i, 1, _Tb), jnp.int32),
            jax.ShapeDtypeStruct((n, _D), jnp.float32),
        ],
        scratch_shapes=[
            pltpu.VMEM((_K, _D), jnp.float32),
            pltpu.VMEM((_Tb, _D), jnp.float32),
            pltpu.VMEM((_Tb, _K), jnp.float32),
        ],
    )(tokens, W, b.reshape(1, _D), cb)
    return idx3.reshape(-1), xn


def _sc_gather2(cb_i, cb_d, idx_i, idx_d):
    info = plsc.get_sparse_core_info()
    nw = info.num_cores * info.num_subcores
    n = idx_i.shape[0]
    bpw = n // nw
    mesh = plsc.VectorSubcoreMesh(core_axis_name="c", subcore_axis_name="s")

    def body(cbi_hbm, cbd_hbm, idxi_hbm, idxd_hbm, qi_hbm, qd_hbm,
             iv1, rv1, iv2, rv2, sem):
        wid = lax.axis_index("s") * info.num_cores + lax.axis_index("c")
        base = wid * bpw
        pltpu.sync_copy(idxi_hbm.at[pl.ds(base, bpw)], iv1)
        h1 = pltpu.async_copy(cbi_hbm.at[iv1], rv1, sem)
        pltpu.sync_copy(idxd_hbm.at[pl.ds(base, bpw)], iv2)
        h2 = pltpu.async_copy(cbd_hbm.at[iv2], rv2, sem)
        h1.wait()
        pltpu.sync_copy(rv1, qi_hbm.at[pl.ds(base, bpw)])
        h2.wait()
        pltpu.sync_copy(rv2, qd_hbm.at[pl.ds(base, bpw)])

    return pl.kernel(
        body, mesh=mesh,
        out_type=[jax.ShapeDtypeStruct((n, _D), jnp.float32),
                  jax.ShapeDtypeStruct((n, _D), jnp.float32)],
        scratch_types=[
            pltpu.VMEM((bpw,), jnp.int32),
            pltpu.VMEM((bpw, _D), jnp.float32),
            pltpu.VMEM((bpw,), jnp.int32),
            pltpu.VMEM((bpw, _D), jnp.float32),
            pltpu.SemaphoreType.DMA,
        ],
    )(cb_i, cb_d, idx_i, idx_d)


def _dec_body(qi_ref, qd_ref, xi_ref, xd_ref, wd_ref, bd_ref,
              out_ref, li_ref, ld_ref):
    i = pl.program_id(0)
    qi = qi_ref[...]
    qi = qi / (jnp.sqrt(jnp.sum(qi * qi, axis=1, keepdims=True)) + _EPS)
    qd = qd_ref[...]
    qd = qd / (jnp.sqrt(jnp.sum(qd * qd, axis=1, keepdims=True)) + _EPS)
    acc = jnp.dot(qi, wd_ref[0:_D, :], preferred_element_type=jnp.float32)
    acc = acc + jnp.dot(qd, wd_ref[_D:2 * _D, :],
                        preferred_element_type=jnp.float32)
    out_ref[...] = acc + bd_ref[...]

    di = qi - xi_ref[...]
    dd = qd - xd_ref[...]
    ti = jnp.sum(di * di).reshape(1, 1)
    td = jnp.sum(dd * dd).reshape(1, 1)

    @pl.when(i == 0)
    def _():
        li_ref[...] = ti
        ld_ref[...] = td

    @pl.when(i != 0)
    def _():
        li_ref[...] = li_ref[...] + ti
        ld_ref[...] = ld_ref[...] + td

    @pl.when(i == pl.num_programs(0) - 1)
    def _():
        n_tok = pl.num_programs(0) * out_ref.shape[0]
        scale = 0.25 / (n_tok * _D)
        li_ref[...] = scale * li_ref[...]
        ld_ref[...] = scale * ld_ref[...]


def _decoder(q_inv, q_dep, xn_inv, xn_dep, W_dec, b_dec):
    n = q_inv.shape[0]
    blk = 512
    z, li, ld = pl.pallas_call(
        _dec_body,
        grid=(n // blk,),
        in_specs=[
            pl.BlockSpec((blk, _D), lambda i: (i, 0)),
            pl.BlockSpec((blk, _D), lambda i: (i, 0)),
            pl.BlockSpec((blk, _D), lambda i: (i, 0)),
            pl.BlockSpec((blk, _D), lambda i: (i, 0)),
            pl.BlockSpec((2 * _D, _L), lambda i: (0, 0)),
            pl.BlockSpec((1, _L), lambda i: (0, 0)),
        ],
        out_specs=[
            pl.BlockSpec((blk, _L), lambda i: (i, 0)),
            pl.BlockSpec((1, 1), lambda i: (0, 0)),
            pl.BlockSpec((1, 1), lambda i: (0, 0)),
        ],
        out_shape=[
            jax.ShapeDtypeStruct((n, _L), jnp.float32),
            jax.ShapeDtypeStruct((1, 1), jnp.float32),
            jax.ShapeDtypeStruct((1, 1), jnp.float32),
        ],
    )(q_inv, q_dep, xn_inv, xn_dep, W_dec, b_dec.reshape(1, _L))
    return z, li[0, 0], ld[0, 0]


def kernel(h_inv_tokens, h_dep_tokens, W_inv, b_inv, W_dep, b_dep,
           cb_inv, cb_dep, W_dec, b_dec):
    B, T, L = h_inv_tokens.shape
    n = B * T
    ti = h_inv_tokens.reshape(n, L)
    td = h_dep_tokens.reshape(n, L)

    idx_i, xn_i = _vq_head(ti, W_inv, b_inv, cb_inv)
    idx_d, xn_d = _vq_head(td, W_dep, b_dep, cb_dep)
    q_i, q_d = _sc_gather2(cb_inv, cb_dep, idx_i, idx_d)

    z, loss_i, loss_d = _decoder(q_i, q_d, xn_i, xn_d, W_dec, b_dec)
    z = z.reshape(B, T, L)
    return z, loss_i, loss_d, idx_i.reshape(B, T), idx_d.reshape(B, T)


# speedup vs baseline: 1.9936x; 1.0002x over previous
import functools

import jax
import jax.numpy as jnp
from jax import lax
from jax.experimental import pallas as pl
from jax.experimental.pallas import tpu as pltpu
from jax.experimental.pallas import tpu_sc as plsc

_L = 768
_D = 256
_K = 8192
_Tb = 512
_Kb = 2048
_NKB = _K // _Kb
_EPS = 1e-8


def _vq_body(tok_ref, w_ref, b_ref, cb_ref, idx_ref, xn_out_ref,
             cbn_ref, sim_ref):
    i = pl.program_id(0)

    @pl.when(i == 0)
    def _():
        cb = cb_ref[...]
        nrm = jnp.sqrt(jnp.sum(cb * cb, axis=1, keepdims=True))
        cbn_ref[...] = cb / (nrm + _EPS)

    h = jnp.dot(tok_ref[...], w_ref[...],
                preferred_element_type=jnp.float32) + b_ref[...]
    nrm = jnp.sqrt(jnp.sum(h * h, axis=1, keepdims=True))
    xn_out_ref[...] = h / (nrm + _EPS)

    sim_ref[...] = lax.dot_general(
        xn_out_ref[...], cbn_ref[...],
        (((1,), (1,)), ((), ())), preferred_element_type=jnp.float32)

    aml = jnp.argmax(sim_ref[...], axis=1).astype(jnp.int32)
    idx_ref[...] = aml.reshape(idx_ref.shape)


def _vq_head(tokens, W, b, cb):
    n = tokens.shape[0]
    grid_i = n // _Tb
    idx3, xn = pl.pallas_call(
        _vq_body,
        grid=(grid_i,),
        in_specs=[
            pl.BlockSpec((_Tb, _L), lambda i: (i, 0)),
            pl.BlockSpec((_L, _D), lambda i: (0, 0)),
            pl.BlockSpec((1, _D), lambda i: (0, 0)),
            pl.BlockSpec((_K, _D), lambda i: (0, 0)),
        ],
        out_specs=[
            pl.BlockSpec((1, 1, _Tb), lambda i: (i, 0, 0)),
            pl.BlockSpec((_Tb, _D), lambda i: (i, 0)),
        ],
        out_shape=[
            jax.ShapeDtypeStruct((grid_i, 1, _Tb), jnp.int32),
            jax.ShapeDtypeStruct((n, _D), jnp.float32),
        ],
        scratch_shapes=[
            pltpu.VMEM((_K, _D), jnp.float32),
            pltpu.VMEM((_Tb, _K), jnp.float32),
        ],
    )(tokens, W, b.reshape(1, _D), cb)
    return idx3.reshape(-1), xn


def _sc_gather2(cb_i, cb_d, idx_i, idx_d):
    info = plsc.get_sparse_core_info()
    nw = info.num_cores * info.num_subcores
    n = idx_i.shape[0]
    bpw = n // nw
    mesh = plsc.VectorSubcoreMesh(core_axis_name="c", subcore_axis_name="s")

    def body(cbi_hbm, cbd_hbm, idxi_hbm, idxd_hbm, qi_hbm, qd_hbm,
             iv1, rv1, iv2, rv2, sem):
        wid = lax.axis_index("s") * info.num_cores + lax.axis_index("c")
        base = wid * bpw
        pltpu.sync_copy(idxi_hbm.at[pl.ds(base, bpw)], iv1)
        h1 = pltpu.async_copy(cbi_hbm.at[iv1], rv1, sem)
        pltpu.sync_copy(idxd_hbm.at[pl.ds(base, bpw)], iv2)
        h2 = pltpu.async_copy(cbd_hbm.at[iv2], rv2, sem)
        h1.wait()
        pltpu.sync_copy(rv1, qi_hbm.at[pl.ds(base, bpw)])
        h2.wait()
        pltpu.sync_copy(rv2, qd_hbm.at[pl.ds(base, bpw)])

    return pl.kernel(
        body, mesh=mesh,
        out_type=[jax.ShapeDtypeStruct((n, _D), jnp.float32),
                  jax.ShapeDtypeStruct((n, _D), jnp.float32)],
        scratch_types=[
            pltpu.VMEM((bpw,), jnp.int32),
            pltpu.VMEM((bpw, _D), jnp.float32),
            pltpu.VMEM((bpw,), jnp.int32),
            pltpu.VMEM((bpw, _D), jnp.float32),
            pltpu.SemaphoreType.DMA,
        ],
    )(cb_i, cb_d, idx_i, idx_d)


def _dec_body(qi_ref, qd_ref, xi_ref, xd_ref, wd_ref, bd_ref,
              out_ref, li_ref, ld_ref):
    i = pl.program_id(0)
    qi = qi_ref[...]
    qi = qi / (jnp.sqrt(jnp.sum(qi * qi, axis=1, keepdims=True)) + _EPS)
    qd = qd_ref[...]
    qd = qd / (jnp.sqrt(jnp.sum(qd * qd, axis=1, keepdims=True)) + _EPS)
    acc = jnp.dot(qi, wd_ref[0:_D, :], preferred_element_type=jnp.float32)
    acc = acc + jnp.dot(qd, wd_ref[_D:2 * _D, :],
                        preferred_element_type=jnp.float32)
    out_ref[...] = acc + bd_ref[...]

    di = qi - xi_ref[...]
    dd = qd - xd_ref[...]
    ti = jnp.sum(di * di).reshape(1, 1)
    td = jnp.sum(dd * dd).reshape(1, 1)

    @pl.when(i == 0)
    def _():
        li_ref[...] = ti
        ld_ref[...] = td

    @pl.when(i != 0)
    def _():
        li_ref[...] = li_ref[...] + ti
        ld_ref[...] = ld_ref[...] + td

    @pl.when(i == pl.num_programs(0) - 1)
    def _():
        n_tok = pl.num_programs(0) * out_ref.shape[0]
        scale = 0.25 / (n_tok * _D)
        li_ref[...] = scale * li_ref[...]
        ld_ref[...] = scale * ld_ref[...]


def _decoder(q_inv, q_dep, xn_inv, xn_dep, W_dec, b_dec):
    n = q_inv.shape[0]
    blk = 512
    z, li, ld = pl.pallas_call(
        _dec_body,
        grid=(n // blk,),
        in_specs=[
            pl.BlockSpec((blk, _D), lambda i: (i, 0)),
            pl.BlockSpec((blk, _D), lambda i: (i, 0)),
            pl.BlockSpec((blk, _D), lambda i: (i, 0)),
            pl.BlockSpec((blk, _D), lambda i: (i, 0)),
            pl.BlockSpec((2 * _D, _L), lambda i: (0, 0)),
            pl.BlockSpec((1, _L), lambda i: (0, 0)),
        ],
        out_specs=[
            pl.BlockSpec((blk, _L), lambda i: (i, 0)),
            pl.BlockSpec((1, 1), lambda i: (0, 0)),
            pl.BlockSpec((1, 1), lambda i: (0, 0)),
        ],
        out_shape=[
            jax.ShapeDtypeStruct((n, _L), jnp.float32),
            jax.ShapeDtypeStruct((1, 1), jnp.float32),
            jax.ShapeDtypeStruct((1, 1), jnp.float32),
        ],
    )(q_inv, q_dep, xn_inv, xn_dep, W_dec, b_dec.reshape(1, _L))
    return z, li[0, 0], ld[0, 0]


def kernel(h_inv_tokens, h_dep_tokens, W_inv, b_inv, W_dep, b_dep,
           cb_inv, cb_dep, W_dec, b_dec):
    B, T, L = h_inv_tokens.shape
    n = B * T
    ti = h_inv_tokens.reshape(n, L)
    td = h_dep_tokens.reshape(n, L)

    idx_i, xn_i = _vq_head(ti, W_inv, b_inv, cb_inv)
    idx_d, xn_d = _vq_head(td, W_dep, b_dep, cb_dep)
    q_i, q_d = _sc_gather2(cb_inv, cb_dep, idx_i, idx_d)

    z, loss_i, loss_d = _decoder(q_i, q_d, xn_i, xn_d, W_dec, b_dec)
    z = z.reshape(B, T, L)
    return z, loss_i, loss_d, idx_i.reshape(B, T), idx_d.reshape(B, T)


# speedup vs baseline: 2.0703x; 1.0385x over previous
import functools

import jax
import jax.numpy as jnp
from jax import lax
from jax.experimental import pallas as pl
from jax.experimental.pallas import tpu as pltpu
from jax.experimental.pallas import tpu_sc as plsc

_L = 768
_D = 256
_K = 8192
_Tb = 1024
_Kb = 2048
_NKB = _K // _Kb
_EPS = 1e-8


def _vq_body(tok_ref, w_ref, b_ref, cb_ref, idx_ref, xn_out_ref,
             cbn_ref, sim_ref):
    i = pl.program_id(0)

    @pl.when(i == 0)
    def _():
        cb = cb_ref[...]
        nrm = jnp.sqrt(jnp.sum(cb * cb, axis=1, keepdims=True))
        cbn_ref[...] = cb / (nrm + _EPS)

    h = jnp.dot(tok_ref[...], w_ref[...],
                preferred_element_type=jnp.float32) + b_ref[...]
    nrm = jnp.sqrt(jnp.sum(h * h, axis=1, keepdims=True))
    xn_out_ref[...] = h / (nrm + _EPS)

    sim_ref[...] = lax.dot_general(
        xn_out_ref[...], cbn_ref[...],
        (((1,), (1,)), ((), ())), preferred_element_type=jnp.float32)

    aml = jnp.argmax(sim_ref[...], axis=1).astype(jnp.int32)
    idx_ref[...] = aml.reshape(idx_ref.shape)


def _vq_head(tokens, W, b, cb):
    n = tokens.shape[0]
    grid_i = n // _Tb
    idx3, xn = pl.pallas_call(
        _vq_body,
        grid=(grid_i,),
        in_specs=[
            pl.BlockSpec((_Tb, _L), lambda i: (i, 0)),
            pl.BlockSpec((_L, _D), lambda i: (0, 0)),
            pl.BlockSpec((1, _D), lambda i: (0, 0)),
            pl.BlockSpec((_K, _D), lambda i: (0, 0)),
        ],
        out_specs=[
            pl.BlockSpec((1, 1, _Tb), lambda i: (i, 0, 0)),
            pl.BlockSpec((_Tb, _D), lambda i: (i, 0)),
        ],
        out_shape=[
            jax.ShapeDtypeStruct((grid_i, 1, _Tb), jnp.int32),
            jax.ShapeDtypeStruct((n, _D), jnp.float32),
        ],
        scratch_shapes=[
            pltpu.VMEM((_K, _D), jnp.float32),
            pltpu.VMEM((_Tb, _K), jnp.float32),
        ],
    )(tokens, W, b.reshape(1, _D), cb)
    return idx3.reshape(-1), xn


def _sc_gather2(cb_i, cb_d, idx_i, idx_d):
    info = plsc.get_sparse_core_info()
    nw = info.num_cores * info.num_subcores
    n = idx_i.shape[0]
    bpw = n // nw
    mesh = plsc.VectorSubcoreMesh(core_axis_name="c", subcore_axis_name="s")

    def body(cbi_hbm, cbd_hbm, idxi_hbm, idxd_hbm, qi_hbm, qd_hbm,
             iv1, rv1, iv2, rv2, sem):
        wid = lax.axis_index("s") * info.num_cores + lax.axis_index("c")
        base = wid * bpw
        pltpu.sync_copy(idxi_hbm.at[pl.ds(base, bpw)], iv1)
        h1 = pltpu.async_copy(cbi_hbm.at[iv1], rv1, sem)
        pltpu.sync_copy(idxd_hbm.at[pl.ds(base, bpw)], iv2)
        h2 = pltpu.async_copy(cbd_hbm.at[iv2], rv2, sem)
        h1.wait()
        pltpu.sync_copy(rv1, qi_hbm.at[pl.ds(base, bpw)])
        h2.wait()
        pltpu.sync_copy(rv2, qd_hbm.at[pl.ds(base, bpw)])

    return pl.kernel(
        body, mesh=mesh,
        out_type=[jax.ShapeDtypeStruct((n, _D), jnp.float32),
                  jax.ShapeDtypeStruct((n, _D), jnp.float32)],
        scratch_types=[
            pltpu.VMEM((bpw,), jnp.int32),
            pltpu.VMEM((bpw, _D), jnp.float32),
            pltpu.VMEM((bpw,), jnp.int32),
            pltpu.VMEM((bpw, _D), jnp.float32),
            pltpu.SemaphoreType.DMA,
        ],
    )(cb_i, cb_d, idx_i, idx_d)


def _dec_body(qi_ref, qd_ref, xi_ref, xd_ref, wd_ref, bd_ref,
              out_ref, li_ref, ld_ref):
    i = pl.program_id(0)
    qi = qi_ref[...]
    qi = qi / (jnp.sqrt(jnp.sum(qi * qi, axis=1, keepdims=True)) + _EPS)
    qd = qd_ref[...]
    qd = qd / (jnp.sqrt(jnp.sum(qd * qd, axis=1, keepdims=True)) + _EPS)
    acc = jnp.dot(qi, wd_ref[0:_D, :], preferred_element_type=jnp.float32)
    acc = acc + jnp.dot(qd, wd_ref[_D:2 * _D, :],
                        preferred_element_type=jnp.float32)
    out_ref[...] = acc + bd_ref[...]

    di = qi - xi_ref[...]
    dd = qd - xd_ref[...]
    ti = jnp.sum(di * di).reshape(1, 1)
    td = jnp.sum(dd * dd).reshape(1, 1)

    @pl.when(i == 0)
    def _():
        li_ref[...] = ti
        ld_ref[...] = td

    @pl.when(i != 0)
    def _():
        li_ref[...] = li_ref[...] + ti
        ld_ref[...] = ld_ref[...] + td

    @pl.when(i == pl.num_programs(0) - 1)
    def _():
        n_tok = pl.num_programs(0) * out_ref.shape[0]
        scale = 0.25 / (n_tok * _D)
        li_ref[...] = scale * li_ref[...]
        ld_ref[...] = scale * ld_ref[...]


def _decoder(q_inv, q_dep, xn_inv, xn_dep, W_dec, b_dec):
    n = q_inv.shape[0]
    blk = 512
    z, li, ld = pl.pallas_call(
        _dec_body,
        grid=(n // blk,),
        in_specs=[
            pl.BlockSpec((blk, _D), lambda i: (i, 0)),
            pl.BlockSpec((blk, _D), lambda i: (i, 0)),
            pl.BlockSpec((blk, _D), lambda i: (i, 0)),
            pl.BlockSpec((blk, _D), lambda i: (i, 0)),
            pl.BlockSpec((2 * _D, _L), lambda i: (0, 0)),
            pl.BlockSpec((1, _L), lambda i: (0, 0)),
        ],
        out_specs=[
            pl.BlockSpec((blk, _L), lambda i: (i, 0)),
            pl.BlockSpec((1, 1), lambda i: (0, 0)),
            pl.BlockSpec((1, 1), lambda i: (0, 0)),
        ],
        out_shape=[
            jax.ShapeDtypeStruct((n, _L), jnp.float32),
            jax.ShapeDtypeStruct((1, 1), jnp.float32),
            jax.ShapeDtypeStruct((1, 1), jnp.float32),
        ],
    )(q_inv, q_dep, xn_inv, xn_dep, W_dec, b_dec.reshape(1, _L))
    return z, li[0, 0], ld[0, 0]


def kernel(h_inv_tokens, h_dep_tokens, W_inv, b_inv, W_dep, b_dep,
           cb_inv, cb_dep, W_dec, b_dec):
    B, T, L = h_inv_tokens.shape
    n = B * T
    ti = h_inv_tokens.reshape(n, L)
    td = h_dep_tokens.reshape(n, L)

    idx_i, xn_i = _vq_head(ti, W_inv, b_inv, cb_inv)
    idx_d, xn_d = _vq_head(td, W_dep, b_dep, cb_dep)
    q_i, q_d = _sc_gather2(cb_inv, cb_dep, idx_i, idx_d)

    z, loss_i, loss_d = _decoder(q_i, q_d, xn_i, xn_d, W_dec, b_dec)
    z = z.reshape(B, T, L)
    return z, loss_i, loss_d, idx_i.reshape(B, T), idx_d.reshape(B, T)
